# baseline (speedup 1.0000x reference)
import functools

import jax
import jax.numpy as jnp
from jax import lax
from jax.experimental import pallas as pl
from jax.experimental.pallas import tpu as pltpu
from jax.experimental.pallas import tpu_sc as plsc

NUM_USERS = 20000
NUM_PRODUCTS = 30000
D = 64
NQ = 4
HQ = 16
N = NUM_USERS + NUM_PRODUCTS
E = 800000
NUM_LAYERS = 3

NC = 2
NS = 16
W = 128
NBUF = 2

N_PAD = 50048
NH = N_PAD // 2
NTRASH = 64
NACC = 25088
TRASH = NH
PAD_IDX = 50000
LAY_WPS = 392
E_PAD = NS * LAY_WPS * W
NR = NACC // NS

FR = N_PAD * HQ // 128
BR = 368
NBLK = FR // BR

_MESH = plsc.VectorSubcoreMesh(core_axis_name="c", subcore_axis_name="s")
_SC_PARAMS = pltpu.CompilerParams(use_tc_tiling_on_sc=False)


@functools.partial(
    pl.kernel,
    out_type=jax.ShapeDtypeStruct((2, NC, NACC, HQ), jnp.float32),
    mesh=_MESH,
    scratch_types=[
        pltpu.VMEM((LAY_WPS, W), jnp.int32),
        pltpu.VMEM((LAY_WPS, W), jnp.int32),
        pltpu.VMEM((NBUF, W, HQ), jnp.float32),
        pltpu.VMEM_SHARED((NACC, HQ), jnp.float32),
        pltpu.SemaphoreType.DMA,
        pltpu.SemaphoreType.DMA,
        pltpu.SemaphoreType.DMA,
        pltpu.SemaphoreType.DMA,
    ],
    compiler_params=_SC_PARAMS,
)
def _sc_pass(y_hbm, row_hbm, col_hbm, a_hbm, rowv, colv, gbuf, acc,
             sem0, sem1, ssem0, ssem1):
    c = lax.axis_index("c")
    s = lax.axis_index("s")
    sems = [sem0, sem1]
    ssems = [ssem0, ssem1]
    pltpu.sync_copy(row_hbm.at[c].at[s], rowv)
    r0 = s * NR
    zrow = jnp.zeros((HQ,), jnp.float32)

    for h in range(2):
        pltpu.sync_copy(col_hbm.at[h].at[s], colv)

        @pl.loop(0, W)
        def _(r):
            gbuf[0, r, :] = zrow

        @pl.loop(0, NR // W)
        def _(k):
            pltpu.sync_copy(gbuf.at[0], acc.at[pl.ds(r0 + k * W, W)])

        if NR % W:
            pltpu.sync_copy(gbuf.at[0].at[pl.ds(0, NR % W)],
                            acc.at[pl.ds(r0 + (NR // W) * W, NR % W)])
        plsc.subcore_barrier()

        for b in range(NBUF):
            pltpu.async_copy(y_hbm.at[rowv.at[b]], gbuf.at[b], sems[b])

        @pl.loop(0, LAY_WPS - NBUF, step=NBUF)
        def _(g):
            for b in range(NBUF):
                pltpu.make_async_copy(
                    y_hbm.at[rowv.at[g + b]], gbuf.at[b], sems[b]).wait()
                pltpu.async_copy(
                    gbuf.at[b], acc.at[colv.at[g + b]], ssems[b], add=True)
            for b in range(NBUF):
                pltpu.make_async_copy(
                    gbuf.at[b], acc.at[colv.at[g + b]], ssems[b]).wait()
                pltpu.async_copy(
                    y_hbm.at[rowv.at[g + b + NBUF]], gbuf.at[b], sems[b])

        for b in range(NBUF):
            g = LAY_WPS - NBUF + b
            pltpu.make_async_copy(
                y_hbm.at[rowv.at[g]], gbuf.at[b], sems[b]).wait()
            pltpu.sync_copy(gbuf.at[b], acc.at[colv.at[g]], add=True)

        plsc.subcore_barrier()
        pltpu.sync_copy(acc.at[pl.ds(r0, NR)],
                        a_hbm.at[h].at[c].at[pl.ds(r0, NR)])


def _dis_body(d_ref, out_ref):
    deg = d_ref[...]
    out_ref[...] = jnp.where(deg > 0.0, lax.rsqrt(deg), 0.0)


def _tc_dis(d):
    return pl.pallas_call(
        _dis_body,
        grid=(NBLK,),
        in_specs=[pl.BlockSpec((BR, 128), lambda i: (i, 0))],
        out_specs=pl.BlockSpec((BR, 128), lambda i: (i, 0)),
        out_shape=jax.ShapeDtypeStruct((FR, 128), jnp.float32),
    )(d.reshape(FR, 128))


def _step_body(scale_a, with_s, a01_ref, a23_ref, dis_ref, *refs):
    if with_s:
        s_ref = refs[0]
        out_refs = refs[1:]
    else:
        out_refs = refs
    dis = dis_ref[...][None]
    a = jnp.concatenate([a01_ref[...], a23_ref[...]], axis=0)
    x = a * dis if scale_a else a
    out_refs[0][...] = s_ref[...] + x if with_s else x
    out_refs[1][...] = x * dis


def _tc_step(a01, a23, dis, s=None, *, scale_a=True):
    with_s = s is not None
    a01 = a01.reshape(NC, FR, 128)
    a23 = a23.reshape(NC, FR, 128)
    half = pl.BlockSpec((2, BR, 128), lambda i: (0, i, 0))
    full = pl.BlockSpec((NQ, BR, 128), lambda i: (0, i, 0))
    dspec = pl.BlockSpec((BR, 128), lambda i: (i, 0))
    in_specs = [half, half, dspec] + ([full] if with_s else [])
    shp = jax.ShapeDtypeStruct((NQ, FR, 128), jnp.float32)
    args = (a01, a23, dis, s) if with_s else (a01, a23, dis)
    return pl.pallas_call(
        functools.partial(_step_body, scale_a, with_s),
        grid=(NBLK,),
        in_specs=in_specs,
        out_specs=[full, full],
        out_shape=[shp, shp],
    )(*args)


@jax.jit
def _impl(edge_index, user_weight, product_weight):
    i32 = jnp.int32
    row = edge_index[0].astype(i32)
    col = edge_index[1].astype(i32)
    pad = jnp.full((E_PAD - E,), PAD_IDX, i32)
    rowp = jnp.concatenate([row, pad])
    colp = jnp.concatenate([col, pad])
    rows2 = jnp.stack([
        (rowp[None, :] + jnp.array([[2 * q], [2 * q + 1]], i32) * N_PAD)
        .reshape(NC, NS, LAY_WPS, W)
        for q in range(2)
    ] + [
        jnp.broadcast_to(
            (jnp.arange(E_PAD, dtype=i32) % 4096)
            .reshape(1, NS, LAY_WPS, W), (NC, NS, LAY_WPS, W)),
    ])
    spread = TRASH + (jnp.arange(E_PAD, dtype=i32) % NTRASH)
    col2 = jnp.stack([
        jnp.where((colp >= h * NH) & (colp < (h + 1) * NH),
                  colp - h * NH, spread).reshape(NS, LAY_WPS, W)
        for h in range(2)
    ])

    xall = jnp.concatenate([user_weight, product_weight], axis=0)
    xall = jnp.concatenate(
        [xall, jnp.zeros((N_PAD - N, D), jnp.float32)], axis=0)
    x01 = jnp.stack([xall[:, 0:16], xall[:, 16:32]])
    x23 = jnp.stack([xall[:, 32:48], xall[:, 48:64]])
    x4 = jnp.concatenate([x01, x23], axis=0)

    def body(carry, i):
        y, ssum, dis, a01k = carry
        q = jnp.where(i == 0, 2, jnp.abs(i - 1) % 2)
        rows = lax.dynamic_index_in_dim(rows2, q, 0, keepdims=False)
        a = _sc_pass(y, rows, col2)
        afull = jnp.concatenate([a[0, :, :NH], a[1, :, :NH]], axis=1)

        def deg_branch():
            d = _tc_dis(afull[0])
            _, y0 = _tc_step(x01, x23, d, scale_a=False)
            return y0.reshape(NQ * N_PAD, HQ), ssum, d, a01k

        def keep_branch():
            return y, ssum, dis, afull

        def step_branch():
            s2, y2 = _tc_step(a01k, afull, dis, ssum)
            return y2.reshape(NQ * N_PAD, HQ), s2, dis, a01k

        bidx = jnp.where(i == 0, 0, jnp.where(i % 2 == 1, 1, 2))
        carry = lax.switch(bidx, [deg_branch, keep_branch, step_branch])
        return carry, None

    init = (
        jnp.ones((NQ * N_PAD, HQ), jnp.float32),
        x4.reshape(NQ, FR, 128),
        jnp.zeros((FR, 128), jnp.float32),
        jnp.zeros((NC, N_PAD, HQ), jnp.float32),
    )
    nsteps = 1 + 2 * NUM_LAYERS
    (_, ssum, _, _), _ = lax.scan(
        body, init, jnp.arange(nsteps, dtype=i32))

    final = ssum.reshape(NQ, N_PAD, HQ) * (1.0 / (NUM_LAYERS + 1))
    user_out = jnp.concatenate(
        [final[qq, :NUM_USERS] for qq in range(NQ)], axis=1)
    product_out = jnp.concatenate(
        [final[qq, NUM_USERS:N] for qq in range(NQ)], axis=1)
    return user_out, product_out


def kernel(edge_index, user_weight, product_weight):
    return _impl(edge_index, user_weight, product_weight)


# speedup vs baseline: 1.0235x; 1.0235x over previous
import functools

import jax
import jax.numpy as jnp
from jax import lax
from jax.experimental import pallas as pl
from jax.experimental.pallas import tpu as pltpu
from jax.experimental.pallas import tpu_sc as plsc

NUM_USERS = 20000
NUM_PRODUCTS = 30000
D = 64
NQ = 4
HQ = 16
N = NUM_USERS + NUM_PRODUCTS
E = 800000
NUM_LAYERS = 3

NC = 2
NS = 16
W = 128
NBUF = 2

N_PAD = 50048
NH = N_PAD // 2
NTRASH = 64
NACC = 25088
TRASH = NH
PAD_IDX = 50000
LAY_WPS = 392
E_PAD = NS * LAY_WPS * W
NR = NACC // NS

FR = N_PAD * HQ // 128
BR = 368
NBLK = FR // BR

_MESH = plsc.VectorSubcoreMesh(core_axis_name="c", subcore_axis_name="s")
_SC_PARAMS = pltpu.CompilerParams(use_tc_tiling_on_sc=False)


@functools.partial(
    pl.kernel,
    out_type=jax.ShapeDtypeStruct((2, NC, NACC, HQ), jnp.float32),
    mesh=_MESH,
    scratch_types=[
        pltpu.VMEM((LAY_WPS, W), jnp.int32),
        pltpu.VMEM((LAY_WPS, W), jnp.int32),
        pltpu.VMEM((NBUF, W, HQ), jnp.float32),
        pltpu.VMEM_SHARED((NACC, HQ), jnp.float32),
        pltpu.SemaphoreType.DMA,
        pltpu.SemaphoreType.DMA,
    ],
    compiler_params=_SC_PARAMS,
)
def _sc_pass(y_hbm, row_hbm, col_hbm, a_hbm, rowv, colv, gbuf, acc,
             sem0, sem1):
    c = lax.axis_index("c")
    s = lax.axis_index("s")
    sems = [sem0, sem1]
    pltpu.sync_copy(row_hbm.at[c].at[s], rowv)
    r0 = s * NR
    zrow = jnp.zeros((HQ,), jnp.float32)

    for h in range(2):
        pltpu.sync_copy(col_hbm.at[h].at[s], colv)

        @pl.loop(0, W)
        def _(r):
            gbuf[0, r, :] = zrow

        @pl.loop(0, NR // W)
        def _(k):
            pltpu.sync_copy(gbuf.at[0], acc.at[pl.ds(r0 + k * W, W)])

        if NR % W:
            pltpu.sync_copy(gbuf.at[0].at[pl.ds(0, NR % W)],
                            acc.at[pl.ds(r0 + (NR // W) * W, NR % W)])
        plsc.subcore_barrier()

        for b in range(NBUF):
            pltpu.async_copy(y_hbm.at[rowv.at[b]], gbuf.at[b], sems[b])

        @pl.loop(0, LAY_WPS - NBUF, step=NBUF)
        def _(g):
            for b in range(NBUF):
                pltpu.make_async_copy(
                    y_hbm.at[rowv.at[g + b]], gbuf.at[b], sems[b]).wait()
                pltpu.sync_copy(gbuf.at[b], acc.at[colv.at[g + b]], add=True)
                pltpu.async_copy(
                    y_hbm.at[rowv.at[g + b + NBUF]], gbuf.at[b], sems[b])

        for b in range(NBUF):
            g = LAY_WPS - NBUF + b
            pltpu.make_async_copy(
                y_hbm.at[rowv.at[g]], gbuf.at[b], sems[b]).wait()
            pltpu.sync_copy(gbuf.at[b], acc.at[colv.at[g]], add=True)

        plsc.subcore_barrier()
        pltpu.sync_copy(acc.at[pl.ds(r0, NR)],
                        a_hbm.at[h].at[c].at[pl.ds(r0, NR)])


def _dis_body(d_ref, out_ref):
    deg = d_ref[...]
    out_ref[...] = jnp.where(deg > 0.0, lax.rsqrt(deg), 0.0)


def _tc_dis(d):
    return pl.pallas_call(
        _dis_body,
        grid=(NBLK,),
        in_specs=[pl.BlockSpec((BR, 128), lambda i: (i, 0))],
        out_specs=pl.BlockSpec((BR, 128), lambda i: (i, 0)),
        out_shape=jax.ShapeDtypeStruct((FR, 128), jnp.float32),
    )(d.reshape(FR, 128))


def _step_body(scale_a, with_s, a01_ref, a23_ref, dis_ref, *refs):
    if with_s:
        s_ref = refs[0]
        out_refs = refs[1:]
    else:
        out_refs = refs
    dis = dis_ref[...][None]
    a = jnp.concatenate([a01_ref[...], a23_ref[...]], axis=0)
    x = a * dis if scale_a else a
    out_refs[0][...] = s_ref[...] + x if with_s else x
    out_refs[1][...] = x * dis


def _tc_step(a01, a23, dis, s=None, *, scale_a=True):
    with_s = s is not None
    a01 = a01.reshape(NC, FR, 128)
    a23 = a23.reshape(NC, FR, 128)
    half = pl.BlockSpec((2, BR, 128), lambda i: (0, i, 0))
    full = pl.BlockSpec((NQ, BR, 128), lambda i: (0, i, 0))
    dspec = pl.BlockSpec((BR, 128), lambda i: (i, 0))
    in_specs = [half, half, dspec] + ([full] if with_s else [])
    shp = jax.ShapeDtypeStruct((NQ, FR, 128), jnp.float32)
    args = (a01, a23, dis, s) if with_s else (a01, a23, dis)
    return pl.pallas_call(
        functools.partial(_step_body, scale_a, with_s),
        grid=(NBLK,),
        in_specs=in_specs,
        out_specs=[full, full],
        out_shape=[shp, shp],
    )(*args)


@jax.jit
def _impl(edge_index, user_weight, product_weight):
    i32 = jnp.int32
    row = edge_index[0].astype(i32)
    col = edge_index[1].astype(i32)
    pad = jnp.full((E_PAD - E,), PAD_IDX, i32)
    rowp = jnp.concatenate([row, pad])
    colp = jnp.concatenate([col, pad])
    rows2 = jnp.stack([
        (rowp[None, :] + jnp.array([[2 * q], [2 * q + 1]], i32) * N_PAD)
        .reshape(NC, NS, LAY_WPS, W)
        for q in range(2)
    ] + [
        jnp.broadcast_to(
            (jnp.arange(E_PAD, dtype=i32) % 4096)
            .reshape(1, NS, LAY_WPS, W), (NC, NS, LAY_WPS, W)),
    ])
    spread = TRASH + (jnp.arange(E_PAD, dtype=i32) % NTRASH)
    col2 = jnp.stack([
        jnp.where((colp >= h * NH) & (colp < (h + 1) * NH),
                  colp - h * NH, spread).reshape(NS, LAY_WPS, W)
        for h in range(2)
    ])

    xall = jnp.concatenate([user_weight, product_weight], axis=0)
    xall = jnp.concatenate(
        [xall, jnp.zeros((N_PAD - N, D), jnp.float32)], axis=0)
    x01 = jnp.stack([xall[:, 0:16], xall[:, 16:32]])
    x23 = jnp.stack([xall[:, 32:48], xall[:, 48:64]])
    x4 = jnp.concatenate([x01, x23], axis=0)

    def body(carry, i):
        y, ssum, dis, a01k = carry
        q = jnp.where(i == 0, 2, jnp.abs(i - 1) % 2)
        rows = lax.dynamic_index_in_dim(rows2, q, 0, keepdims=False)
        a = _sc_pass(y, rows, col2)
        afull = jnp.concatenate([a[0, :, :NH], a[1, :, :NH]], axis=1)

        def deg_branch():
            d = _tc_dis(afull[0])
            _, y0 = _tc_step(x01, x23, d, scale_a=False)
            return y0.reshape(NQ * N_PAD, HQ), ssum, d, a01k

        def keep_branch():
            return y, ssum, dis, afull

        def step_branch():
            s2, y2 = _tc_step(a01k, afull, dis, ssum)
            return y2.reshape(NQ * N_PAD, HQ), s2, dis, a01k

        bidx = jnp.where(i == 0, 0, jnp.where(i % 2 == 1, 1, 2))
        carry = lax.switch(bidx, [deg_branch, keep_branch, step_branch])
        return carry, None

    init = (
        jnp.ones((NQ * N_PAD, HQ), jnp.float32),
        x4.reshape(NQ, FR, 128),
        jnp.zeros((FR, 128), jnp.float32),
        jnp.zeros((NC, N_PAD, HQ), jnp.float32),
    )
    nsteps = 1 + 2 * NUM_LAYERS
    (_, ssum, _, _), _ = lax.scan(
        body, init, jnp.arange(nsteps, dtype=i32))

    final = ssum.reshape(NQ, N_PAD, HQ) * (1.0 / (NUM_LAYERS + 1))
    user_out = jnp.concatenate(
        [final[qq, :NUM_USERS] for qq in range(NQ)], axis=1)
    product_out = jnp.concatenate(
        [final[qq, NUM_USERS:N] for qq in range(NQ)], axis=1)
    return user_out, product_out


def kernel(edge_index, user_weight, product_weight):
    return _impl(edge_index, user_weight, product_weight)
